# Initial kernel scaffold; baseline (speedup 1.0000x reference)
import jax
import jax.numpy as jnp
from jax.experimental import pallas as pl


def kernel(edge_index, edge_type, subj, rel, emb_h, emb_e, W_e, b_e, rel_wt, w_rel, bn0_g, bn0_b, bn1_g, bn1_b, concat_W, concat_b, bnc_g, bnc_b):
    raise NotImplementedError("write your pallas kernel here")


# speedup vs baseline: 4.5368x; 4.5368x over previous
import functools

import jax
import jax.numpy as jnp
from jax import lax
from jax.experimental import pallas as pl
from jax.experimental.pallas import tpu as pltpu
from jax.experimental.pallas import tpu_sc as plsc

N_ENT = 10000
E = 320000
D = 128
NUM_REL = 101
B = 1024

NC = 2
NS = 16
L = 16
NW = NC * NS

EROW = 32
EPT = 10240
E_PAD = NW * EPT
PAD_E = E_PAD - E
RPT = EPT // EROW
CH = 32
NCHK = RPT // CH

N_PAD = 10240
ZROW = 32
NZ = N_PAD // ZROW // NS

_mesh = plsc.VectorSubcoreMesh(
    core_axis_name="c", subcore_axis_name="s", num_cores=NC, num_subcores=NS)


def _zero_rows(buf, nrows, ncols):
    def body(i, _):
        for j in range(ncols // L):
            buf[i, pl.ds(j * L, L)] = jnp.zeros((L,), jnp.float32)
        return 0
    lax.fori_loop(0, nrows, body, 0)


def _make_sc_pass(with_rel):
    nf = 3 if with_rel else 2
    nbuf = 2 if with_rel else 4

    scratch = [pltpu.VMEM((CH * nf, EROW), jnp.int32)]
    scratch += [pltpu.VMEM((EROW, D), jnp.float32) for _ in range(nbuf)]
    if with_rel:
        scratch += [pltpu.VMEM((EROW, D), jnp.float32) for _ in range(4)]
    scratch += [pltpu.VMEM_SHARED((N_PAD, D), jnp.float32)]
    if with_rel:
        scratch += [pltpu.VMEM_SHARED((NUM_REL, D), jnp.float32)]
    nsem = 6 if with_rel else 8
    scratch += [pltpu.SemaphoreType.DMA for _ in range(nsem)]

    def body(*refs):
        if with_rel:
            (tab_hbm, rel_hbm, pidx_hbm, out_hbm, pidx,
             s0, s1, r0, r1, o0, o1, agg, rel_sh, *sems) = refs
            srows = (s0, s1)
            rrows = (r0, r1)
            obuf = (o0, o1)
            gse = sems[0:2]
            gsr = sems[2:4]
            ssc = sems[4:6]
        else:
            (tab_hbm, pidx_hbm, out_hbm, pidx,
             s0, s1, s2, s3, agg, *sems) = refs
            srows = (s0, s1, s2, s3)
            obuf = srows
            gse = sems[0:4]
            ssc = sems[4:8]

        c = lax.axis_index("c")
        s = lax.axis_index("s")
        wid = s * NC + c

        _zero_rows(s0, EROW, D)

        def zc(k, _):
            chunk = s + k * NS
            pltpu.sync_copy(s0, agg.at[pl.ds(chunk * ZROW, ZROW)])
            return 0
        lax.fori_loop(0, NZ, zc, 0)

        if with_rel:
            @pl.when(s == 0)
            def _():
                pltpu.sync_copy(rel_hbm, rel_sh)

        plsc.subcore_barrier()

        def fire_g(j, u):
            pltpu.async_copy(tab_hbm.at[pidx.at[nf * j]], srows[u], gse[u])
            if with_rel:
                pltpu.async_copy(
                    rel_sh.at[pidx.at[nf * j + 1]], rrows[u], gsr[u])

        def wait_g(j, u):
            pltpu.make_async_copy(
                tab_hbm.at[pidx.at[nf * j]], srows[u], gse[u]).wait()
            if with_rel:
                pltpu.make_async_copy(
                    rel_sh.at[pidx.at[nf * j + 1]], rrows[u], gsr[u]).wait()

        def fire_sc(j, u):
            pltpu.async_copy(
                obuf[u], agg.at[pidx.at[nf * j + nf - 1]], ssc[u], add=True)

        def wait_sc(j, u):
            pltpu.make_async_copy(
                obuf[u], agg.at[pidx.at[nf * j + nf - 1]], ssc[u]).wait()

        def mul(u):
            def mbody(i2, _):
                for v in range(2):
                    for jj in range(D // L):
                        r = 2 * i2 + v
                        sl = pl.ds(jj * L, L)
                        obuf[u][r, sl] = srows[u][r, sl] * rrows[u][r, sl]
                return 0
            lax.fori_loop(0, EROW // 2, mbody, 0)

        if with_rel:
            def chunk_body():
                def pair(k, _):
                    for u in range(2):
                        j = 2 * k + u
                        wait_g(j, u)

                        @pl.when(k > 0)
                        def _():
                            wait_sc(j - 2, u)
                        mul(u)
                        fire_sc(j, u)

                        @pl.when(k < CH // 2 - 1)
                        def _():
                            fire_g(j + 2, u)
                    return 0
                lax.fori_loop(0, CH // 2, pair, 0)
        else:
            def chunk_body():
                def quad(k, _):
                    for u in range(4):
                        j = 4 * k + u
                        wait_g(j, u)
                        fire_sc(j, u)
                        w = (u + 2) % 4
                        if u < 2:
                            @pl.when(k > 0)
                            def _():
                                wait_sc(j - 2, w)
                            fire_g(j + 2, w)
                        else:
                            wait_sc(j - 2, w)

                            @pl.when(k < CH // 4 - 1)
                            def _():
                                fire_g(j + 2, w)
                    return 0
                lax.fori_loop(0, CH // 4, quad, 0)

        npend = 2
        pend_slots = (0, 1) if with_rel else (2, 3)
        for t in range(NCHK):
            if t > 0:
                for z, u in enumerate(pend_slots):
                    wait_sc(CH - npend + z, u)
            pltpu.sync_copy(pidx_hbm.at[wid * NCHK + t], pidx)
            fire_g(0, 0)
            fire_g(1, 1)
            chunk_body()
        for z, u in enumerate(pend_slots):
            wait_sc(CH - npend + z, u)

        plsc.subcore_barrier()

        def wb(k, _):
            chunk = s + k * NS
            sl = pl.ds(chunk * ZROW, ZROW)
            pltpu.sync_copy(agg.at[sl], out_hbm.at[c, sl])
            return 0
        lax.fori_loop(0, NZ, wb, 0)

    return pl.kernel(
        body,
        out_type=jax.ShapeDtypeStruct((NC, N_PAD, D), jnp.float32),
        mesh=_mesh,
        scratch_types=scratch,
    )


_sc_msg_pass = _make_sc_pass(with_rel=True)
_sc_agg_pass = _make_sc_pass(with_rel=False)


def _bn_relu(x, g, b):
    mu = jnp.mean(x, axis=0, keepdims=True)
    var = jnp.mean((x - mu) ** 2, axis=0, keepdims=True)
    return jnp.maximum((x - mu) / jnp.sqrt(var + 1e-5) * g + b, 0.0)


def _tc_proj_body(emb_h_ref, w_e_ref, b_e_ref, rel_wt_ref, emb_e_ref,
                  ent_out, rel_out):
    ent_out[...] = (
        jnp.dot(emb_h_ref[...], w_e_ref[...], preferred_element_type=jnp.float32)
        + b_e_ref[...]
    )
    rel_out[...] = jnp.dot(
        rel_wt_ref[...], emb_e_ref[...], preferred_element_type=jnp.float32
    )


def _tc_bn0_body(p_ref, ent_ref, relrow_ref, g_ref, b_ref, out_ref):
    agg = (p_ref[0, :N_ENT, :] + p_ref[1, :N_ENT, :]
           + ent_ref[...] * relrow_ref[...])
    out_ref[...] = _bn_relu(agg, g_ref[...], b_ref[...])


def _tc_head_body(p_ref, z_ref, rel_e_ref, w_rel_ref, subj_ref,
                  rel_ref, wtop_ref, wbot_ref, cb_ref, g1_ref, b1_ref,
                  gc_ref, bc_ref, h_out, q_out):
    z = z_ref[...]
    agg1 = p_ref[0, :N_ENT, :] + p_ref[1, :N_ENT, :] + z
    h1 = _bn_relu(agg1, g1_ref[...], b1_ref[...])
    hc = (
        jnp.dot(z, wtop_ref[...], preferred_element_type=jnp.float32)
        + jnp.dot(h1, wbot_ref[...], preferred_element_type=jnp.float32)
        + cb_ref[...]
    )
    h = _bn_relu(hc, gc_ref[...], bc_ref[...])
    h_out[...] = h

    rel2 = jnp.dot(rel_e_ref[...], w_rel_ref[...], preferred_element_type=jnp.float32)
    ohr = (rel_ref[...] == lax.broadcasted_iota(jnp.int32, (B, NUM_REL), 1))
    q_r = jnp.dot(ohr.astype(jnp.float32), rel2, preferred_element_type=jnp.float32)

    subj = subj_ref[...]
    acc = jnp.zeros((B, D), jnp.float32)
    blk = 2000
    for k in range(N_ENT // blk):
        iota = lax.broadcasted_iota(jnp.int32, (B, blk), 1) + k * blk
        oh = (subj == iota).astype(jnp.float32)
        acc = acc + jnp.dot(oh, h[k * blk:(k + 1) * blk, :],
                            preferred_element_type=jnp.float32)
    q_out[...] = acc * q_r


def _tc_score_body(q_ref, h_ref, out_ref):
    out_ref[...] = lax.dot_general(
        q_ref[...], h_ref[...],
        (((1,), (1,)), ((), ())),
        preferred_element_type=jnp.float32,
    )


def kernel(edge_index, edge_type, subj, rel, emb_h, emb_e, W_e, b_e, rel_wt,
           w_rel, bn0_g, bn0_b, bn1_g, bn1_b, concat_W, concat_b, bnc_g, bnc_b):
    i32 = jnp.int32
    src_f = jnp.concatenate([edge_index[0].astype(i32), jnp.zeros((PAD_E,), i32)])
    dst_f = jnp.concatenate([edge_index[1].astype(i32), jnp.full((PAD_E,), N_ENT, i32)])
    et_f = jnp.concatenate([edge_type.astype(i32), jnp.zeros((PAD_E,), i32)])

    def _pack(arrs):
        parts = [a.reshape(NW, NCHK, CH, 1, EROW) for a in arrs]
        return jnp.concatenate(parts, axis=3).reshape(
            NW * NCHK, CH * len(arrs), EROW)

    pidx1 = _pack([src_f, et_f, dst_f])
    pidx2 = _pack([src_f, dst_f])
    subj2d = subj.astype(i32).reshape(B, 1)
    rel2d = rel.astype(i32).reshape(B, 1)

    ent, rel_embed = pl.pallas_call(
        _tc_proj_body,
        out_shape=(
            jax.ShapeDtypeStruct((N_ENT, D), jnp.float32),
            jax.ShapeDtypeStruct((NUM_REL, D), jnp.float32),
        ),
    )(emb_h, W_e, b_e.reshape(1, D), rel_wt, emb_e)

    p1 = _sc_msg_pass(ent, rel_embed, pidx1)

    zero_out = pl.pallas_call(
        _tc_bn0_body,
        out_shape=jax.ShapeDtypeStruct((N_ENT, D), jnp.float32),
    )(p1, ent, rel_embed[NUM_REL - 1:NUM_REL], bn0_g.reshape(1, D),
      bn0_b.reshape(1, D))

    p2 = _sc_agg_pass(zero_out, pidx2)

    h, q = pl.pallas_call(
        _tc_head_body,
        out_shape=(
            jax.ShapeDtypeStruct((N_ENT, D), jnp.float32),
            jax.ShapeDtypeStruct((B, D), jnp.float32),
        ),
    )(p2, zero_out, rel_embed, w_rel, subj2d, rel2d,
      concat_W[:D], concat_W[D:], concat_b.reshape(1, D),
      bn1_g.reshape(1, D), bn1_b.reshape(1, D),
      bnc_g.reshape(1, D), bnc_b.reshape(1, D))

    score = pl.pallas_call(
        _tc_score_body,
        out_shape=jax.ShapeDtypeStruct((B, N_ENT), jnp.float32),
    )(q, h)
    return score


# speedup vs baseline: 4.8297x; 1.0645x over previous
import functools

import jax
import jax.numpy as jnp
from jax import lax
from jax.experimental import pallas as pl
from jax.experimental.pallas import tpu as pltpu
from jax.experimental.pallas import tpu_sc as plsc

N_ENT = 10000
E = 320000
D = 128
NUM_REL = 101
B = 1024

NC = 2
NS = 16
L = 16
NW = NC * NS

EROW = 64
EPT = 10240
E_PAD = NW * EPT
PAD_E = E_PAD - E
RPT = EPT // EROW
CH = 16
NCHK = RPT // CH

N_PAD = 10240
ZROW = 64
NZ = N_PAD // ZROW // NS

_mesh = plsc.VectorSubcoreMesh(
    core_axis_name="c", subcore_axis_name="s", num_cores=NC, num_subcores=NS)


def _zero_rows(buf, nrows, ncols):
    def body(i, _):
        for j in range(ncols // L):
            buf[i, pl.ds(j * L, L)] = jnp.zeros((L,), jnp.float32)
        return 0
    lax.fori_loop(0, nrows, body, 0)


def _make_sc_pass(with_rel):
    nf = 3 if with_rel else 2
    nbuf = 2 if with_rel else 4

    scratch = [pltpu.VMEM((CH * nf, EROW), jnp.int32)]
    scratch += [pltpu.VMEM((EROW, D), jnp.float32) for _ in range(nbuf)]
    if with_rel:
        scratch += [pltpu.VMEM((EROW, D), jnp.float32) for _ in range(2)]
    scratch += [pltpu.VMEM_SHARED((N_PAD, D), jnp.float32)]
    if with_rel:
        scratch += [pltpu.VMEM_SHARED((NUM_REL, D), jnp.float32)]
    nsem = 6 if with_rel else 8
    scratch += [pltpu.SemaphoreType.DMA for _ in range(nsem)]

    def body(*refs):
        if with_rel:
            (tab_hbm, rel_hbm, pidx_hbm, out_hbm, pidx,
             s0, s1, r0, r1, agg, rel_sh, *sems) = refs
            srows = (s0, s1)
            rrows = (r0, r1)
            obuf = srows
            gse = sems[0:2]
            gsr = sems[2:4]
            ssc = sems[4:6]
        else:
            (tab_hbm, pidx_hbm, out_hbm, pidx,
             s0, s1, s2, s3, agg, *sems) = refs
            srows = (s0, s1, s2, s3)
            obuf = srows
            gse = sems[0:4]
            ssc = sems[4:8]

        c = lax.axis_index("c")
        s = lax.axis_index("s")
        wid = s * NC + c

        _zero_rows(s0, EROW, D)

        def zc(k, _):
            chunk = s + k * NS
            pltpu.sync_copy(s0, agg.at[pl.ds(chunk * ZROW, ZROW)])
            return 0
        lax.fori_loop(0, NZ, zc, 0)

        if with_rel:
            @pl.when(s == 0)
            def _():
                pltpu.sync_copy(rel_hbm, rel_sh)

        plsc.subcore_barrier()

        def fire_g(j, u):
            pltpu.async_copy(tab_hbm.at[pidx.at[nf * j]], srows[u], gse[u])
            if with_rel:
                pltpu.async_copy(
                    rel_sh.at[pidx.at[nf * j + 1]], rrows[u], gsr[u])

        def wait_g(j, u):
            pltpu.make_async_copy(
                tab_hbm.at[pidx.at[nf * j]], srows[u], gse[u]).wait()
            if with_rel:
                pltpu.make_async_copy(
                    rel_sh.at[pidx.at[nf * j + 1]], rrows[u], gsr[u]).wait()

        def fire_sc(j, u):
            pltpu.async_copy(
                obuf[u], agg.at[pidx.at[nf * j + nf - 1]], ssc[u], add=True)

        def wait_sc(j, u):
            pltpu.make_async_copy(
                obuf[u], agg.at[pidx.at[nf * j + nf - 1]], ssc[u]).wait()

        def mul(u):
            def mbody(i2, _):
                for v in range(2):
                    for jj in range(D // L):
                        r = 2 * i2 + v
                        sl = pl.ds(jj * L, L)
                        obuf[u][r, sl] = srows[u][r, sl] * rrows[u][r, sl]
                return 0
            lax.fori_loop(0, EROW // 2, mbody, 0)

        if with_rel:
            def chunk_body():
                def pair(k, _):
                    j0 = 2 * k
                    wait_g(j0, 0)

                    @pl.when(k > 0)
                    def _():
                        wait_sc(j0 - 1, 1)
                        fire_g(j0 + 1, 1)
                    mul(0)
                    fire_sc(j0, 0)

                    wait_g(j0 + 1, 1)
                    wait_sc(j0, 0)

                    @pl.when(k < CH // 2 - 1)
                    def _():
                        fire_g(j0 + 2, 0)
                    mul(1)
                    fire_sc(j0 + 1, 1)
                    return 0
                lax.fori_loop(0, CH // 2, pair, 0)
        else:
            def chunk_body():
                def quad(k, _):
                    for u in range(4):
                        j = 4 * k + u
                        wait_g(j, u)
                        fire_sc(j, u)
                        w = (u + 2) % 4
                        if u < 2:
                            @pl.when(k > 0)
                            def _():
                                wait_sc(j - 2, w)
                            fire_g(j + 2, w)
                        else:
                            wait_sc(j - 2, w)

                            @pl.when(k < CH // 4 - 1)
                            def _():
                                fire_g(j + 2, w)
                    return 0
                lax.fori_loop(0, CH // 4, quad, 0)

        pend = ((CH - 1, 1),) if with_rel else ((CH - 2, 2), (CH - 1, 3))
        for t in range(NCHK):
            if t > 0:
                for j, u in pend:
                    wait_sc(j, u)
            pltpu.sync_copy(pidx_hbm.at[wid * NCHK + t], pidx)
            fire_g(0, 0)
            fire_g(1, 1)
            chunk_body()
        for j, u in pend:
            wait_sc(j, u)

        plsc.subcore_barrier()

        def wb(k, _):
            chunk = s + k * NS
            sl = pl.ds(chunk * ZROW, ZROW)
            pltpu.sync_copy(agg.at[sl], out_hbm.at[c, sl])
            return 0
        lax.fori_loop(0, NZ, wb, 0)

    return pl.kernel(
        body,
        out_type=jax.ShapeDtypeStruct((NC, N_PAD, D), jnp.float32),
        mesh=_mesh,
        scratch_types=scratch,
    )


_sc_msg_pass = _make_sc_pass(with_rel=True)
_sc_agg_pass = _make_sc_pass(with_rel=False)


def _bn_relu(x, g, b):
    mu = jnp.mean(x, axis=0, keepdims=True)
    var = jnp.mean((x - mu) ** 2, axis=0, keepdims=True)
    return jnp.maximum((x - mu) / jnp.sqrt(var + 1e-5) * g + b, 0.0)


def _tc_proj_body(emb_h_ref, w_e_ref, b_e_ref, rel_wt_ref, emb_e_ref,
                  ent_out, rel_out):
    ent_out[...] = (
        jnp.dot(emb_h_ref[...], w_e_ref[...], preferred_element_type=jnp.float32)
        + b_e_ref[...]
    )
    rel_out[...] = jnp.dot(
        rel_wt_ref[...], emb_e_ref[...], preferred_element_type=jnp.float32
    )


def _tc_bn0_body(p_ref, ent_ref, relrow_ref, g_ref, b_ref, out_ref):
    agg = (p_ref[0, :N_ENT, :] + p_ref[1, :N_ENT, :]
           + ent_ref[...] * relrow_ref[...])
    out_ref[...] = _bn_relu(agg, g_ref[...], b_ref[...])


def _tc_head_body(p_ref, z_ref, rel_e_ref, w_rel_ref, subj_ref,
                  rel_ref, wtop_ref, wbot_ref, cb_ref, g1_ref, b1_ref,
                  gc_ref, bc_ref, h_out, q_out):
    z = z_ref[...]
    agg1 = p_ref[0, :N_ENT, :] + p_ref[1, :N_ENT, :] + z
    h1 = _bn_relu(agg1, g1_ref[...], b1_ref[...])
    hc = (
        jnp.dot(z, wtop_ref[...], preferred_element_type=jnp.float32)
        + jnp.dot(h1, wbot_ref[...], preferred_element_type=jnp.float32)
        + cb_ref[...]
    )
    h = _bn_relu(hc, gc_ref[...], bc_ref[...])
    h_out[...] = h

    rel2 = jnp.dot(rel_e_ref[...], w_rel_ref[...], preferred_element_type=jnp.float32)
    ohr = (rel_ref[...] == lax.broadcasted_iota(jnp.int32, (B, NUM_REL), 1))
    q_r = jnp.dot(ohr.astype(jnp.float32), rel2, preferred_element_type=jnp.float32)

    subj = subj_ref[...]
    acc = jnp.zeros((B, D), jnp.float32)
    blk = 2000
    for k in range(N_ENT // blk):
        iota = lax.broadcasted_iota(jnp.int32, (B, blk), 1) + k * blk
        oh = (subj == iota).astype(jnp.float32)
        acc = acc + jnp.dot(oh, h[k * blk:(k + 1) * blk, :],
                            preferred_element_type=jnp.float32)
    q_out[...] = acc * q_r


def _tc_score_body(q_ref, h_ref, out_ref):
    out_ref[...] = lax.dot_general(
        q_ref[...], h_ref[...],
        (((1,), (1,)), ((), ())),
        preferred_element_type=jnp.float32,
    )


def kernel(edge_index, edge_type, subj, rel, emb_h, emb_e, W_e, b_e, rel_wt,
           w_rel, bn0_g, bn0_b, bn1_g, bn1_b, concat_W, concat_b, bnc_g, bnc_b):
    i32 = jnp.int32
    src_f = jnp.concatenate([edge_index[0].astype(i32), jnp.zeros((PAD_E,), i32)])
    dst_f = jnp.concatenate([edge_index[1].astype(i32), jnp.full((PAD_E,), N_ENT, i32)])
    et_f = jnp.concatenate([edge_type.astype(i32), jnp.zeros((PAD_E,), i32)])

    def _pack(arrs):
        parts = [a.reshape(NW, NCHK, CH, 1, EROW) for a in arrs]
        return jnp.concatenate(parts, axis=3).reshape(
            NW * NCHK, CH * len(arrs), EROW)

    pidx1 = _pack([src_f, et_f, dst_f])
    pidx2 = _pack([src_f, dst_f])
    subj2d = subj.astype(i32).reshape(B, 1)
    rel2d = rel.astype(i32).reshape(B, 1)

    ent, rel_embed = pl.pallas_call(
        _tc_proj_body,
        out_shape=(
            jax.ShapeDtypeStruct((N_ENT, D), jnp.float32),
            jax.ShapeDtypeStruct((NUM_REL, D), jnp.float32),
        ),
    )(emb_h, W_e, b_e.reshape(1, D), rel_wt, emb_e)

    p1 = _sc_msg_pass(ent, rel_embed, pidx1)

    zero_out = pl.pallas_call(
        _tc_bn0_body,
        out_shape=jax.ShapeDtypeStruct((N_ENT, D), jnp.float32),
    )(p1, ent, rel_embed[NUM_REL - 1:NUM_REL], bn0_g.reshape(1, D),
      bn0_b.reshape(1, D))

    p2 = _sc_agg_pass(zero_out, pidx2)

    h, q = pl.pallas_call(
        _tc_head_body,
        out_shape=(
            jax.ShapeDtypeStruct((N_ENT, D), jnp.float32),
            jax.ShapeDtypeStruct((B, D), jnp.float32),
        ),
    )(p2, zero_out, rel_embed, w_rel, subj2d, rel2d,
      concat_W[:D], concat_W[D:], concat_b.reshape(1, D),
      bn1_g.reshape(1, D), bn1_b.reshape(1, D),
      bnc_g.reshape(1, D), bnc_b.reshape(1, D))

    score = pl.pallas_call(
        _tc_score_body,
        out_shape=jax.ShapeDtypeStruct((B, N_ENT), jnp.float32),
    )(q, h)
    return score
